# baseline (speedup 1.0000x reference)
import dataclasses
import functools

import jax
import jax.numpy as jnp
from jax import lax
from jax.experimental import pallas as pl
from jax.experimental.pallas import tpu as pltpu
from jax.experimental.pallas import tpu_sc as plsc

B = 16384
D = 32
PACK = 128 // D
B_ROWS = 1000000 // PACK
NC = 2
NS = 16
NW = NC * NS
BPW = B // NW
CH = 128
NCH = BPW // CH
L = 16
GPC = CH // L


def _mf_body(uid_hbm, mid_hbm, ru_hbm, rm_hbm, cu_hbm, cm_hbm,
             p2_hbm, q2_hbm, bu_hbm, bi_hbm, out_hbm,
             uid_v, mid_v, ru_v, rm_v, cu_v, cm_v,
             pu_v, qi_v, bu_v, bi_v, out_v,
             sem_a, sem_b, sem_i):
    wid = lax.axis_index("s") * NC + lax.axis_index("c")
    base = wid * BPW
    bsl = pl.ds(base, BPW)

    idx_copies = [
        pltpu.async_copy(uid_hbm.at[bsl], uid_v, sem_i),
        pltpu.async_copy(mid_hbm.at[bsl], mid_v, sem_i),
        pltpu.async_copy(ru_hbm.at[bsl], ru_v, sem_i),
        pltpu.async_copy(rm_hbm.at[bsl], rm_v, sem_i),
        pltpu.async_copy(cu_hbm.at[bsl], cu_v, sem_i),
        pltpu.async_copy(cm_hbm.at[bsl], cm_v, sem_i),
    ]
    for cp in idx_copies:
        cp.wait()

    bias_copies = []
    for c in range(NCH):
        sl = pl.ds(c * CH, CH)
        bias_copies.append(pltpu.async_copy(bu_hbm.at[uid_v.at[sl]],
                                            bu_v.at[sl], sem_b))
        bias_copies.append(pltpu.async_copy(bi_hbm.at[mid_v.at[sl]],
                                            bi_v.at[sl], sem_b))

    sems = (sem_a, sem_b)

    def fire(c):
        sl = pl.ds(c * CH, CH)
        buf = c % 2
        sem = sems[buf]
        return [pltpu.async_copy(p2_hbm.at[ru_v.at[sl]], pu_v.at[buf], sem),
                pltpu.async_copy(q2_hbm.at[rm_v.at[sl]], qi_v.at[buf], sem)]

    lane = lax.iota(jnp.int32, L)
    pending = fire(0)
    for cb in bias_copies:
        cb.wait()
    for c in range(NCH):
        nxt = fire(c + 1) if c + 1 < NCH else []
        for cp in pending:
            cp.wait()
        pending = nxt
        buf = c % 2
        pu_b, qi_b = pu_v.at[buf], qi_v.at[buf]
        for g in range(GPC):
            j = c * CH + g * L
            jsl = pl.ds(j, L)
            rows = g * L + lane
            cu = cu_v[jsl]
            cm = cm_v[jsl]
            acc = bu_v[jsl] + bi_v[jsl]
            for k in range(D):
                acc = acc + (plsc.load_gather(pu_b, [rows, cu + k]) *
                             plsc.load_gather(qi_b, [rows, cm + k]))
            out_v[jsl] = acc

    pltpu.sync_copy(out_v, out_hbm.at[bsl])


@jax.jit
def kernel(user_id, movie_id, P, Q, b_u, b_i):
    uid = user_id.astype(jnp.int32)
    mid = movie_id.astype(jnp.int32)
    p2 = P.reshape(B_ROWS, 128)
    q2 = Q.reshape(B_ROWS, 128)
    ru = uid >> 2
    rm = mid >> 2
    cu = (uid & 3) * D
    cm = (mid & 3) * D
    mesh = plsc.VectorSubcoreMesh(core_axis_name="c", subcore_axis_name="s",
                                  num_cores=NC, num_subcores=NS)
    cp = pltpu.CompilerParams()
    if "needs_layout_passes" in pltpu.CompilerParams.__dataclass_fields__:
        cp = dataclasses.replace(cp, needs_layout_passes=False)
    mf = pl.kernel(
        _mf_body,
        out_type=jax.ShapeDtypeStruct((B,), jnp.float32),
        mesh=mesh,
        scratch_types=[
            pltpu.VMEM((BPW,), jnp.int32),
            pltpu.VMEM((BPW,), jnp.int32),
            pltpu.VMEM((BPW,), jnp.int32),
            pltpu.VMEM((BPW,), jnp.int32),
            pltpu.VMEM((BPW,), jnp.int32),
            pltpu.VMEM((BPW,), jnp.int32),
            pltpu.VMEM((2, CH, 128), jnp.float32),
            pltpu.VMEM((2, CH, 128), jnp.float32),
            pltpu.VMEM((BPW,), jnp.float32),
            pltpu.VMEM((BPW,), jnp.float32),
            pltpu.VMEM((BPW,), jnp.float32),
            pltpu.SemaphoreType.DMA,
            pltpu.SemaphoreType.DMA,
            pltpu.SemaphoreType.DMA,
        ],
        compiler_params=cp,
    )
    return mf(uid, mid, ru, rm, cu, cm, p2, q2,
              b_u.reshape(-1), b_i.reshape(-1))


# speedup vs baseline: 1.0006x; 1.0006x over previous
import dataclasses
import functools

import jax
import jax.numpy as jnp
from jax import lax
from jax.experimental import pallas as pl
from jax.experimental.pallas import tpu as pltpu
from jax.experimental.pallas import tpu_sc as plsc

B = 16384
D = 32
PACK = 128 // D
B_ROWS = 1000000 // PACK
NC = 2
NS = 16
NW = NC * NS
BPW = B // NW
CH = 128
NCH = BPW // CH
L = 16
GPC = CH // L


def _mf_body(uid_hbm, mid_hbm, ru_hbm, rm_hbm, cu_hbm, cm_hbm,
             p2_hbm, q2_hbm, bu_hbm, bi_hbm, out_hbm,
             uid_v, mid_v, ru_v, rm_v, cu_v, cm_v,
             pu_v, qi_v, bu_v, bi_v, out_v,
             sem_a, sem_b, sem_i):
    wid = lax.axis_index("s") * NC + lax.axis_index("c")
    base = wid * BPW
    bsl = pl.ds(base, BPW)

    idx_copies = [
        pltpu.async_copy(uid_hbm.at[bsl], uid_v, sem_i),
        pltpu.async_copy(mid_hbm.at[bsl], mid_v, sem_i),
        pltpu.async_copy(ru_hbm.at[bsl], ru_v, sem_i),
        pltpu.async_copy(rm_hbm.at[bsl], rm_v, sem_i),
        pltpu.async_copy(cu_hbm.at[bsl], cu_v, sem_i),
        pltpu.async_copy(cm_hbm.at[bsl], cm_v, sem_i),
    ]
    for cp in idx_copies:
        cp.wait()

    bias_copies = []
    for c in range(NCH):
        sl = pl.ds(c * CH, CH)
        bias_copies.append(pltpu.async_copy(bu_hbm.at[uid_v.at[sl]],
                                            bu_v.at[sl], sem_b))
        bias_copies.append(pltpu.async_copy(bi_hbm.at[mid_v.at[sl]],
                                            bi_v.at[sl], sem_b))

    sems = (sem_a, sem_b)

    def fire(c):
        sl = pl.ds(c * CH, CH)
        buf = c % 2
        sem = sems[buf]
        return [pltpu.async_copy(p2_hbm.at[ru_v.at[sl]], pu_v.at[buf], sem),
                pltpu.async_copy(q2_hbm.at[rm_v.at[sl]], qi_v.at[buf], sem)]

    lane = lax.iota(jnp.int32, L)
    pending = fire(0)
    for cb in bias_copies:
        cb.wait()
    for c in range(NCH):
        nxt = fire(c + 1) if c + 1 < NCH else []
        for cp in pending:
            cp.wait()
        pending = nxt
        buf = c % 2
        pu_b, qi_b = pu_v.at[buf], qi_v.at[buf]
        for g in range(GPC):
            j = c * CH + g * L
            jsl = pl.ds(j, L)
            rows = g * L + lane
            cu = cu_v[jsl]
            cm = cm_v[jsl]
            acc = bu_v[jsl] + bi_v[jsl]
            for k in range(D):
                acc = acc + (plsc.load_gather(pu_b, [rows, cu + k]) *
                             plsc.load_gather(qi_b, [rows, cm + k]))
            out_v[jsl] = acc

    pltpu.sync_copy(out_v, out_hbm.at[bsl])


@jax.jit
def kernel(user_id, movie_id, P, Q, b_u, b_i):
    uid = user_id.astype(jnp.int32)
    mid = movie_id.astype(jnp.int32)
    p2 = (P * jnp.float32(1.0)).reshape(B_ROWS, 128)
    q2 = (Q * jnp.float32(1.0)).reshape(B_ROWS, 128)
    ru = uid >> 2
    rm = mid >> 2
    cu = (uid & 3) * D
    cm = (mid & 3) * D
    mesh = plsc.VectorSubcoreMesh(core_axis_name="c", subcore_axis_name="s",
                                  num_cores=NC, num_subcores=NS)
    cp = pltpu.CompilerParams()
    if "needs_layout_passes" in pltpu.CompilerParams.__dataclass_fields__:
        cp = dataclasses.replace(cp, needs_layout_passes=False)
    mf = pl.kernel(
        _mf_body,
        out_type=jax.ShapeDtypeStruct((B,), jnp.float32),
        mesh=mesh,
        scratch_types=[
            pltpu.VMEM((BPW,), jnp.int32),
            pltpu.VMEM((BPW,), jnp.int32),
            pltpu.VMEM((BPW,), jnp.int32),
            pltpu.VMEM((BPW,), jnp.int32),
            pltpu.VMEM((BPW,), jnp.int32),
            pltpu.VMEM((BPW,), jnp.int32),
            pltpu.VMEM((2, CH, 128), jnp.float32),
            pltpu.VMEM((2, CH, 128), jnp.float32),
            pltpu.VMEM((BPW,), jnp.float32),
            pltpu.VMEM((BPW,), jnp.float32),
            pltpu.VMEM((BPW,), jnp.float32),
            pltpu.SemaphoreType.DMA,
            pltpu.SemaphoreType.DMA,
            pltpu.SemaphoreType.DMA,
        ],
        compiler_params=cp,
    )
    return mf(uid, mid, ru, rm, cu, cm, p2, q2,
              b_u.reshape(-1), b_i.reshape(-1))
